# baseline (speedup 1.0000x reference)
import functools

import jax
import jax.numpy as jnp
from jax import lax
from jax.experimental import pallas as pl
from jax.experimental.pallas import tpu as pltpu
from jax.experimental.pallas import tpu_sc as plsc

N = 10000
D = 128
OUT = 64
E = 320000

NC = 2
NS = 16
NW = NC * NS
CHUNK = 64
NCH = 160
G = 16
TOTAL_CH = NW * NCH
EP = TOTAL_CH * CHUNK
CH_T0 = 272
CH_T1 = 2 * NCH - CH_T0
NP = 10240
ROWS_PER_TILE = NP // NS
DUMMY = N
BR = 1024


def _mesh():
    return plsc.VectorSubcoreMesh(core_axis_name="c", subcore_axis_name="s")


def _degree_kernel(std, dtd, sdt, ddt, zeros_h, ones_h, out, acc, idx_v,
                   ones_v):
    c = lax.axis_index("c")
    s = lax.axis_index("s")
    wid = s * NC + c
    rslice = pl.ds(s * ROWS_PER_TILE, ROWS_PER_TILE)

    pltpu.sync_copy(ones_h, ones_v)

    for a, arr in enumerate((std, dtd, sdt, ddt)):
        pltpu.sync_copy(zeros_h.at[rslice], acc.at[rslice])
        plsc.subcore_barrier()

        def group(g, _):
            pltpu.sync_copy(arr.at[pl.ds(wid * NCH + g * G, G)], idx_v)
            for k in range(G):
                pltpu.sync_copy(ones_v, acc.at[idx_v.at[k]], add=True)
            return 0
        lax.fori_loop(0, NCH // G, group, 0)
        plsc.subcore_barrier()
        pltpu.sync_copy(acc.at[rslice], out.at[c, a, rslice])
        plsc.subcore_barrier()


def _degrees(std, dtd, sdt, ddt, zeros_h, ones_h):
    return pl.kernel(
        _degree_kernel,
        out_type=jax.ShapeDtypeStruct((NC, 4, NP, D), jnp.float32),
        mesh=_mesh(),
        scratch_types=[
            pltpu.VMEM_SHARED((NP, D), jnp.float32),
            pltpu.VMEM((G, CHUNK), jnp.int32),
            pltpu.VMEM((CHUNK, D), jnp.float32),
        ],
    )(std, dtd, sdt, ddt, zeros_h, ones_h)

def _conv_kernel(h_td, src_td, dst_td, h_dt, src_dt, dst_dt, zeros_h,
                 out_td, out_dt, acc, src_v, dst_v, rows_a, rows_b, rows_c,
                 sem_a, sem_b, sem_c):
    c = lax.axis_index("c")
    s = lax.axis_index("s")
    rslice = pl.ds(s * ROWS_PER_TILE, ROWS_PER_TILE)
    base = jnp.where(c == 0, s * CH_T0, NS * CH_T0 + s * CH_T1)
    n_groups = jnp.where(c == 0, CH_T0 // G, CH_T1 // G)

    bufs = (rows_a, rows_b, rows_c)
    sems = (sem_a, sem_b, sem_c)

    def run(h_hbm, src_hbm, dst_hbm, out_hbm):
        pltpu.sync_copy(zeros_h.at[rslice], acc.at[rslice])
        plsc.subcore_barrier()

        def group(g, _):
            pltpu.sync_copy(src_hbm.at[pl.ds(base + g * G, G)], src_v)
            pltpu.sync_copy(dst_hbm.at[pl.ds(base + g * G, G)], dst_v)
            pltpu.async_copy(h_hbm.at[src_v.at[0]], bufs[0], sems[0])
            pltpu.async_copy(h_hbm.at[src_v.at[1]], bufs[1], sems[1])
            for k in range(G):
                pltpu.make_async_copy(h_hbm.at[src_v.at[k]], bufs[k % 3],
                                      sems[k % 3]).wait()
                if k + 2 < G:
                    pltpu.async_copy(h_hbm.at[src_v.at[k + 2]],
                                     bufs[(k + 2) % 3], sems[(k + 2) % 3])
                pltpu.sync_copy(bufs[k % 3], acc.at[dst_v.at[k]], add=True)
            return 0

        lax.fori_loop(0, n_groups, group, 0)
        plsc.subcore_barrier()
        pltpu.sync_copy(acc.at[rslice], out_hbm.at[c, rslice])
        plsc.subcore_barrier()

    run(h_td, src_td, dst_td, out_td)
    run(h_dt, src_dt, dst_dt, out_dt)


def _conv_layer(h_td, src_td, dst_td, h_dt, src_dt, dst_dt, zeros_h):
    return pl.kernel(
        _conv_kernel,
        out_type=(
            jax.ShapeDtypeStruct((NC, NP, D), jnp.float32),
            jax.ShapeDtypeStruct((NC, NP, D), jnp.float32),
        ),
        mesh=_mesh(),
        scratch_types=[
            pltpu.VMEM_SHARED((NP, D), jnp.float32),
            pltpu.VMEM((G, CHUNK), jnp.int32),
            pltpu.VMEM((G, CHUNK), jnp.int32),
            pltpu.VMEM((CHUNK, D), jnp.float32),
            pltpu.VMEM((CHUNK, D), jnp.float32),
            pltpu.VMEM((CHUNK, D), jnp.float32),
            pltpu.SemaphoreType.DMA,
            pltpu.SemaphoreType.DMA,
            pltpu.SemaphoreType.DMA,
        ],
    )(h_td, src_td, dst_td, h_dt, src_dt, dst_dt, zeros_h)


def _tc1_body(hists, xdr, xds, w0td, w0dt, isd_o, h0td_o, h0dt_o):
    deg = jnp.sum(hists[...], axis=0)
    isd = lax.rsqrt(jnp.maximum(deg, 1.0))
    isd_o[...] = isd
    h0td_o[...] = (
        jnp.dot(xdr[...], w0td[...], preferred_element_type=jnp.float32)
        * isd[0][:, None])
    h0dt_o[...] = (
        jnp.dot(xds[...], w0dt[...], preferred_element_type=jnp.float32)
        * isd[2][:, None])


def _tc1(hists, xdr, xds, w0td, w0dt):
    g = NP // BR
    return pl.pallas_call(
        _tc1_body,
        grid=(g,),
        in_specs=[
            pl.BlockSpec((NC, 4, BR), lambda i: (0, 0, i)),
            pl.BlockSpec((BR, D), lambda i: (i, 0)),
            pl.BlockSpec((BR, D), lambda i: (i, 0)),
            pl.BlockSpec((D, D), lambda i: (0, 0)),
            pl.BlockSpec((D, D), lambda i: (0, 0)),
        ],
        out_specs=[
            pl.BlockSpec((4, BR), lambda i: (0, i)),
            pl.BlockSpec((BR, D), lambda i: (i, 0)),
            pl.BlockSpec((BR, D), lambda i: (i, 0)),
        ],
        out_shape=[
            jax.ShapeDtypeStruct((4, NP), jnp.float32),
            jax.ShapeDtypeStruct((NP, D), jnp.float32),
            jax.ShapeDtypeStruct((NP, D), jnp.float32),
        ],
    )(hists, xdr, xds, w0td, w0dt)


def _tc2_body(atd, adt, isd, b0td, b0dt, w1td, w1dt, h1td_o, h1dt_o):
    i = isd[...]
    xs1 = jax.nn.relu((atd[0] + atd[1]) * i[1][:, None] + b0td[...])
    xd1 = jax.nn.relu((adt[0] + adt[1]) * i[3][:, None] + b0dt[...])
    h1td_o[...] = (
        jnp.dot(xd1, w1td[...], preferred_element_type=jnp.float32)
        * i[0][:, None])
    h1dt_o[...] = (
        jnp.dot(xs1, w1dt[...], preferred_element_type=jnp.float32)
        * i[2][:, None])


def _tc2(atd, adt, isd, b0td, b0dt, w1td, w1dt):
    g = NP // BR
    return pl.pallas_call(
        _tc2_body,
        grid=(g,),
        in_specs=[
            pl.BlockSpec((NC, BR, D), lambda i: (0, i, 0)),
            pl.BlockSpec((NC, BR, D), lambda i: (0, i, 0)),
            pl.BlockSpec((4, BR), lambda i: (0, i)),
            pl.BlockSpec((1, D), lambda i: (0, 0)),
            pl.BlockSpec((1, D), lambda i: (0, 0)),
            pl.BlockSpec((D, D), lambda i: (0, 0)),
            pl.BlockSpec((D, D), lambda i: (0, 0)),
        ],
        out_specs=[
            pl.BlockSpec((BR, D), lambda i: (i, 0)),
            pl.BlockSpec((BR, D), lambda i: (i, 0)),
        ],
        out_shape=[
            jax.ShapeDtypeStruct((NP, D), jnp.float32),
            jax.ShapeDtypeStruct((NP, D), jnp.float32),
        ],
    )(atd, adt, isd, b0td, b0dt, w1td, w1dt)


def _tc3_body(atd, adt, isd, b1td, b1dt, linw, linb, dr_o, ds_o):
    i = isd[...]
    xs2 = jax.nn.relu((atd[0] + atd[1]) * i[1][:, None] + b1td[...])
    xd2 = jax.nn.relu((adt[0] + adt[1]) * i[3][:, None] + b1dt[...])
    dr_o[...] = (
        jnp.dot(xd2, linw[...], preferred_element_type=jnp.float32)
        + linb[...])
    ds_o[...] = (
        jnp.dot(xs2, linw[...], preferred_element_type=jnp.float32)
        + linb[...])


def _tc3(atd, adt, isd, b1td, b1dt, linw, linb):
    g = NP // BR
    return pl.pallas_call(
        _tc3_body,
        grid=(g,),
        in_specs=[
            pl.BlockSpec((NC, BR, D), lambda i: (0, i, 0)),
            pl.BlockSpec((NC, BR, D), lambda i: (0, i, 0)),
            pl.BlockSpec((4, BR), lambda i: (0, i)),
            pl.BlockSpec((1, D), lambda i: (0, 0)),
            pl.BlockSpec((1, D), lambda i: (0, 0)),
            pl.BlockSpec((D, OUT), lambda i: (0, 0)),
            pl.BlockSpec((1, OUT), lambda i: (0, 0)),
        ],
        out_specs=[
            pl.BlockSpec((BR, OUT), lambda i: (i, 0)),
            pl.BlockSpec((BR, OUT), lambda i: (i, 0)),
        ],
        out_shape=[
            jax.ShapeDtypeStruct((NP, OUT), jnp.float32),
            jax.ShapeDtypeStruct((NP, OUT), jnp.float32),
        ],
    )(atd, adt, isd, b1td, b1dt, linw, linb)


def _prep_idx(row):
    row = row.astype(jnp.int32)
    row = jnp.pad(row, (0, EP - E), constant_values=DUMMY)
    return row.reshape(TOTAL_CH, CHUNK)


def kernel(x_drug, x_disease, edge_index_td, edge_index_dt,
           W0_td, b0_td, W0_dt, b0_dt,
           W1_td, b1_td, W1_dt, b1_dt,
           lin_W, lin_b):
    std = _prep_idx(edge_index_td[0])
    dtd = _prep_idx(edge_index_td[1])
    sdt = _prep_idx(edge_index_dt[0])
    ddt = _prep_idx(edge_index_dt[1])
    xdr = jnp.pad(x_drug, ((0, NP - N), (0, 0)))
    xds = jnp.pad(x_disease, ((0, NP - N), (0, 0)))
    zeros_h = jnp.zeros((NP, D), jnp.float32)
    ones_h = jnp.ones((CHUNK, D), jnp.float32)

    hists = _degrees(std, dtd, sdt, ddt, zeros_h, ones_h)[:, :, :, 0]
    isd, h0td, h0dt = _tc1(hists, xdr, xds, W0_td, W0_dt)
    atd0, adt0 = _conv_layer(h0td, std, dtd, h0dt, sdt, ddt, zeros_h)
    h1td, h1dt = _tc2(atd0, adt0, isd, b0_td[None, :], b0_dt[None, :],
                      W1_td, W1_dt)
    atd1, adt1 = _conv_layer(h1td, std, dtd, h1dt, sdt, ddt, zeros_h)
    dr, ds = _tc3(atd1, adt1, isd, b1_td[None, :], b1_dt[None, :],
                  lin_W, lin_b[None, :])
    return (dr[:N], ds[:N])


# speedup vs baseline: 1.0628x; 1.0628x over previous
import functools

import jax
import jax.numpy as jnp
from jax import lax
from jax.experimental import pallas as pl
from jax.experimental.pallas import tpu as pltpu
from jax.experimental.pallas import tpu_sc as plsc

N = 10000
D = 128
OUT = 64
E = 320000

NC = 2
NS = 16
NW = NC * NS
CHUNK = 64
NCH = 160
G = 16
TOTAL_CH = NW * NCH
EP = TOTAL_CH * CHUNK
CH_T0 = 288
CH_T1 = 2 * NCH - CH_T0
NP = 10240
ROWS_PER_TILE = NP // NS
DUMMY = N
BR = 1024


def _mesh():
    return plsc.VectorSubcoreMesh(core_axis_name="c", subcore_axis_name="s")


def _degree_kernel(std, dtd, sdt, ddt, zeros_h, ones_h, out, acc, idx_v,
                   ones_v):
    c = lax.axis_index("c")
    s = lax.axis_index("s")
    wid = s * NC + c
    rslice = pl.ds(s * ROWS_PER_TILE, ROWS_PER_TILE)

    pltpu.sync_copy(ones_h, ones_v)

    for a, arr in enumerate((std, dtd, sdt, ddt)):
        pltpu.sync_copy(zeros_h.at[rslice], acc.at[rslice])
        plsc.subcore_barrier()

        def group(g, _):
            pltpu.sync_copy(arr.at[pl.ds(wid * NCH + g * G, G)], idx_v)
            for k in range(G):
                pltpu.sync_copy(ones_v, acc.at[idx_v.at[k]], add=True)
            return 0
        lax.fori_loop(0, NCH // G, group, 0)
        plsc.subcore_barrier()
        pltpu.sync_copy(acc.at[rslice], out.at[c, a, rslice])
        plsc.subcore_barrier()


def _degrees(std, dtd, sdt, ddt, zeros_h, ones_h):
    return pl.kernel(
        _degree_kernel,
        out_type=jax.ShapeDtypeStruct((NC, 4, NP, D), jnp.float32),
        mesh=_mesh(),
        scratch_types=[
            pltpu.VMEM_SHARED((NP, D), jnp.float32),
            pltpu.VMEM((G, CHUNK), jnp.int32),
            pltpu.VMEM((CHUNK, D), jnp.float32),
        ],
    )(std, dtd, sdt, ddt, zeros_h, ones_h)

def _conv_kernel(h_td, src_td, dst_td, h_dt, src_dt, dst_dt, zeros_h,
                 out_td, out_dt, acc, src_v, dst_v, rows_a, rows_b, rows_c,
                 sem_a, sem_b, sem_c):
    c = lax.axis_index("c")
    s = lax.axis_index("s")
    rslice = pl.ds(s * ROWS_PER_TILE, ROWS_PER_TILE)
    base = jnp.where(c == 0, s * CH_T0, NS * CH_T0 + s * CH_T1)
    n_groups = jnp.where(c == 0, CH_T0 // G, CH_T1 // G)

    bufs = (rows_a, rows_b, rows_c)
    sems = (sem_a, sem_b, sem_c)

    def run(h_hbm, src_hbm, dst_hbm, out_hbm):
        pltpu.sync_copy(zeros_h.at[rslice], acc.at[rslice])
        plsc.subcore_barrier()

        def group(g, _):
            pltpu.sync_copy(src_hbm.at[pl.ds(base + g * G, G)], src_v)
            pltpu.sync_copy(dst_hbm.at[pl.ds(base + g * G, G)], dst_v)
            pltpu.async_copy(h_hbm.at[src_v.at[0]], bufs[0], sems[0])
            pltpu.async_copy(h_hbm.at[src_v.at[1]], bufs[1], sems[1])
            for k in range(G):
                pltpu.make_async_copy(h_hbm.at[src_v.at[k]], bufs[k % 3],
                                      sems[k % 3]).wait()
                if k + 2 < G:
                    pltpu.async_copy(h_hbm.at[src_v.at[k + 2]],
                                     bufs[(k + 2) % 3], sems[(k + 2) % 3])
                pltpu.sync_copy(bufs[k % 3], acc.at[dst_v.at[k]], add=True)
            return 0

        lax.fori_loop(0, n_groups, group, 0)
        plsc.subcore_barrier()
        pltpu.sync_copy(acc.at[rslice], out_hbm.at[c, rslice])
        plsc.subcore_barrier()

    run(h_td, src_td, dst_td, out_td)
    run(h_dt, src_dt, dst_dt, out_dt)


def _conv_layer(h_td, src_td, dst_td, h_dt, src_dt, dst_dt, zeros_h):
    return pl.kernel(
        _conv_kernel,
        out_type=(
            jax.ShapeDtypeStruct((NC, NP, D), jnp.float32),
            jax.ShapeDtypeStruct((NC, NP, D), jnp.float32),
        ),
        mesh=_mesh(),
        scratch_types=[
            pltpu.VMEM_SHARED((NP, D), jnp.float32),
            pltpu.VMEM((G, CHUNK), jnp.int32),
            pltpu.VMEM((G, CHUNK), jnp.int32),
            pltpu.VMEM((CHUNK, D), jnp.float32),
            pltpu.VMEM((CHUNK, D), jnp.float32),
            pltpu.VMEM((CHUNK, D), jnp.float32),
            pltpu.SemaphoreType.DMA,
            pltpu.SemaphoreType.DMA,
            pltpu.SemaphoreType.DMA,
        ],
    )(h_td, src_td, dst_td, h_dt, src_dt, dst_dt, zeros_h)


def _tc1_body(hists, xdr, xds, w0td, w0dt, isd_o, h0td_o, h0dt_o):
    deg = jnp.sum(hists[...], axis=0)
    isd = lax.rsqrt(jnp.maximum(deg, 1.0))
    isd_o[...] = isd
    h0td_o[...] = (
        jnp.dot(xdr[...], w0td[...], preferred_element_type=jnp.float32)
        * isd[0][:, None])
    h0dt_o[...] = (
        jnp.dot(xds[...], w0dt[...], preferred_element_type=jnp.float32)
        * isd[2][:, None])


def _tc1(hists, xdr, xds, w0td, w0dt):
    g = NP // BR
    return pl.pallas_call(
        _tc1_body,
        grid=(g,),
        in_specs=[
            pl.BlockSpec((NC, 4, BR), lambda i: (0, 0, i)),
            pl.BlockSpec((BR, D), lambda i: (i, 0)),
            pl.BlockSpec((BR, D), lambda i: (i, 0)),
            pl.BlockSpec((D, D), lambda i: (0, 0)),
            pl.BlockSpec((D, D), lambda i: (0, 0)),
        ],
        out_specs=[
            pl.BlockSpec((4, BR), lambda i: (0, i)),
            pl.BlockSpec((BR, D), lambda i: (i, 0)),
            pl.BlockSpec((BR, D), lambda i: (i, 0)),
        ],
        out_shape=[
            jax.ShapeDtypeStruct((4, NP), jnp.float32),
            jax.ShapeDtypeStruct((NP, D), jnp.float32),
            jax.ShapeDtypeStruct((NP, D), jnp.float32),
        ],
    )(hists, xdr, xds, w0td, w0dt)


def _tc2_body(atd, adt, isd, b0td, b0dt, w1td, w1dt, h1td_o, h1dt_o):
    i = isd[...]
    xs1 = jax.nn.relu((atd[0] + atd[1]) * i[1][:, None] + b0td[...])
    xd1 = jax.nn.relu((adt[0] + adt[1]) * i[3][:, None] + b0dt[...])
    h1td_o[...] = (
        jnp.dot(xd1, w1td[...], preferred_element_type=jnp.float32)
        * i[0][:, None])
    h1dt_o[...] = (
        jnp.dot(xs1, w1dt[...], preferred_element_type=jnp.float32)
        * i[2][:, None])


def _tc2(atd, adt, isd, b0td, b0dt, w1td, w1dt):
    g = NP // BR
    return pl.pallas_call(
        _tc2_body,
        grid=(g,),
        in_specs=[
            pl.BlockSpec((NC, BR, D), lambda i: (0, i, 0)),
            pl.BlockSpec((NC, BR, D), lambda i: (0, i, 0)),
            pl.BlockSpec((4, BR), lambda i: (0, i)),
            pl.BlockSpec((1, D), lambda i: (0, 0)),
            pl.BlockSpec((1, D), lambda i: (0, 0)),
            pl.BlockSpec((D, D), lambda i: (0, 0)),
            pl.BlockSpec((D, D), lambda i: (0, 0)),
        ],
        out_specs=[
            pl.BlockSpec((BR, D), lambda i: (i, 0)),
            pl.BlockSpec((BR, D), lambda i: (i, 0)),
        ],
        out_shape=[
            jax.ShapeDtypeStruct((NP, D), jnp.float32),
            jax.ShapeDtypeStruct((NP, D), jnp.float32),
        ],
    )(atd, adt, isd, b0td, b0dt, w1td, w1dt)


def _tc3_body(atd, adt, isd, b1td, b1dt, linw, linb, dr_o, ds_o):
    i = isd[...]
    xs2 = jax.nn.relu((atd[0] + atd[1]) * i[1][:, None] + b1td[...])
    xd2 = jax.nn.relu((adt[0] + adt[1]) * i[3][:, None] + b1dt[...])
    dr_o[...] = (
        jnp.dot(xd2, linw[...], preferred_element_type=jnp.float32)
        + linb[...])
    ds_o[...] = (
        jnp.dot(xs2, linw[...], preferred_element_type=jnp.float32)
        + linb[...])


def _tc3(atd, adt, isd, b1td, b1dt, linw, linb):
    g = NP // BR
    return pl.pallas_call(
        _tc3_body,
        grid=(g,),
        in_specs=[
            pl.BlockSpec((NC, BR, D), lambda i: (0, i, 0)),
            pl.BlockSpec((NC, BR, D), lambda i: (0, i, 0)),
            pl.BlockSpec((4, BR), lambda i: (0, i)),
            pl.BlockSpec((1, D), lambda i: (0, 0)),
            pl.BlockSpec((1, D), lambda i: (0, 0)),
            pl.BlockSpec((D, OUT), lambda i: (0, 0)),
            pl.BlockSpec((1, OUT), lambda i: (0, 0)),
        ],
        out_specs=[
            pl.BlockSpec((BR, OUT), lambda i: (i, 0)),
            pl.BlockSpec((BR, OUT), lambda i: (i, 0)),
        ],
        out_shape=[
            jax.ShapeDtypeStruct((NP, OUT), jnp.float32),
            jax.ShapeDtypeStruct((NP, OUT), jnp.float32),
        ],
    )(atd, adt, isd, b1td, b1dt, linw, linb)


def _prep_idx(row):
    row = row.astype(jnp.int32)
    row = jnp.pad(row, (0, EP - E), constant_values=DUMMY)
    return row.reshape(TOTAL_CH, CHUNK)


def kernel(x_drug, x_disease, edge_index_td, edge_index_dt,
           W0_td, b0_td, W0_dt, b0_dt,
           W1_td, b1_td, W1_dt, b1_dt,
           lin_W, lin_b):
    std = _prep_idx(edge_index_td[0])
    dtd = _prep_idx(edge_index_td[1])
    sdt = _prep_idx(edge_index_dt[0])
    ddt = _prep_idx(edge_index_dt[1])
    xdr = jnp.pad(x_drug, ((0, NP - N), (0, 0)))
    xds = jnp.pad(x_disease, ((0, NP - N), (0, 0)))
    zeros_h = jnp.zeros((NP, D), jnp.float32)
    ones_h = jnp.ones((CHUNK, D), jnp.float32)

    hists = _degrees(std, dtd, sdt, ddt, zeros_h, ones_h)[:, :, :, 0]
    isd, h0td, h0dt = _tc1(hists, xdr, xds, W0_td, W0_dt)
    atd0, adt0 = _conv_layer(h0td, std, dtd, h0dt, sdt, ddt, zeros_h)
    h1td, h1dt = _tc2(atd0, adt0, isd, b0_td[None, :], b0_dt[None, :],
                      W1_td, W1_dt)
    atd1, adt1 = _conv_layer(h1td, std, dtd, h1dt, sdt, ddt, zeros_h)
    dr, ds = _tc3(atd1, adt1, isd, b1_td[None, :], b1_dt[None, :],
                  lin_W, lin_b[None, :])
    return (dr[:N], ds[:N])
